# Initial kernel scaffold; baseline (speedup 1.0000x reference)
import jax
import jax.numpy as jnp
from jax.experimental import pallas as pl


def kernel(user_emb, item_emb, edge_index, edge_values, W1, b1, W2, b2):
    raise NotImplementedError("write your pallas kernel here")


# speedup vs baseline: 1.6681x; 1.6681x over previous
import functools

import jax
import jax.numpy as jnp
from jax import lax
from jax.experimental import pallas as pl
from jax.experimental.pallas import tpu as pltpu
from jax.experimental.pallas import tpu_sc as plsc

_EMB = 128
_HID = 64
_B_TEMP = 0.5
_NC = 2
_NS = 16
_NT = _NC * _NS
_C = 128


def _tc_tables_body(ne_ref, w1_ref, b1_ref, p_ref, q_ref):
    blk = ne_ref[...]
    w1a = w1_ref[0:_EMB, :]
    w1b = w1_ref[_EMB : 2 * _EMB, :]
    p_ref[...] = (
        jnp.dot(blk, w1a, preferred_element_type=jnp.float32) + b1_ref[...]
    )
    q_ref[...] = jnp.dot(blk, w1b, preferred_element_type=jnp.float32)


def _make_tables(node_emb, W1, b1):
    n_nodes = node_emb.shape[0]
    blk = 1000
    grid = n_nodes // blk
    return pl.pallas_call(
        _tc_tables_body,
        grid=(grid,),
        in_specs=[
            pl.BlockSpec((blk, _EMB), lambda i: (i, 0)),
            pl.BlockSpec((2 * _EMB, _HID), lambda i: (0, 0)),
            pl.BlockSpec((1, _HID), lambda i: (0, 0)),
        ],
        out_specs=[
            pl.BlockSpec((blk, _HID), lambda i: (i, 0)),
            pl.BlockSpec((blk, _HID), lambda i: (i, 0)),
        ],
        out_shape=[
            jax.ShapeDtypeStruct((n_nodes, _HID), jnp.float32),
            jax.ShapeDtypeStruct((n_nodes, _HID), jnp.float32),
        ],
    )(node_emb, W1, b1.reshape(1, _HID))


def _make_sc_kernel(chunks):
    mesh = plsc.VectorSubcoreMesh(core_axis_name="c", subcore_axis_name="s")

    @functools.partial(
        pl.kernel,
        out_type=[
            jax.ShapeDtypeStruct((_NT, chunks, _C), jnp.float32),
            jax.ShapeDtypeStruct((_NT, 16), jnp.float32),
        ],
        mesh=mesh,
        compiler_params=pltpu.CompilerParams(
            needs_layout_passes=False, use_tc_tiling_on_sc=False
        ),
        scratch_types=[
            pltpu.VMEM((chunks, _C), jnp.int32),
            pltpu.VMEM((chunks, _C), jnp.int32),
            pltpu.VMEM((chunks, _C), jnp.float32),
            pltpu.VMEM((chunks, _C), jnp.float32),
            pltpu.VMEM((_C, _HID), jnp.float32),
            pltpu.VMEM((_C,), jnp.float32),
            pltpu.VMEM((_HID,), jnp.float32),
            pltpu.VMEM((16,), jnp.float32),
            pltpu.SemaphoreType.DMA,
        ],
    )
    def sc_edges(
        p_hbm, q_hbm, src_hbm, dst_hbm, ev_hbm, n2_hbm, w2_hbm,
        out_hbm, acc_hbm,
        srct, dstt, evt, n2t, pqbuf, obuf, w2t, accv, sem,
    ):
        cid = lax.axis_index("c")
        sid = lax.axis_index("s")
        wid = sid * _NC + cid
        pltpu.sync_copy(src_hbm.at[wid], srct)
        pltpu.sync_copy(dst_hbm.at[wid], dstt)
        pltpu.sync_copy(ev_hbm.at[wid], evt)
        pltpu.sync_copy(n2_hbm.at[wid], n2t)
        pltpu.sync_copy(w2_hbm, w2t)
        w2regs = [w2t[pl.ds(16 * k, 16)] for k in range(_HID // 16)]
        lanes = lax.iota(jnp.int32, 16)

        def chunk_body(ci, acc):
            pltpu.async_copy(p_hbm.at[srct.at[ci]], pqbuf, sem).wait()
            pltpu.async_copy(q_hbm.at[dstt.at[ci]], pqbuf, sem, add=True).wait()

            def grp_body(gi, acc):
                base = gi * 16
                rows = lanes + base
                lv = jnp.zeros((16,), jnp.float32)
                for j in range(_HID):
                    col = jnp.full((16,), j, jnp.int32)
                    pq = plsc.load_gather(pqbuf, [rows, col])
                    w2s = w2regs[j // 16][j % 16]
                    lv = lv + jnp.maximum(pq, 0.0) * w2s
                s = pl.ds(base, 16)
                x = n2t[ci, s] + lv
                g = 1.0 / (1.0 + jnp.exp(-x))
                acc = acc + g
                obuf[s] = evt[ci, s] * g
                return acc

            acc = lax.fori_loop(0, _C // 16, grp_body, acc)
            pltpu.sync_copy(obuf, out_hbm.at[wid, ci])
            return acc

        acc = lax.fori_loop(
            0, chunks, chunk_body, jnp.zeros((16,), jnp.float32)
        )
        accv[...] = acc
        pltpu.sync_copy(accv, acc_hbm.at[wid])

    return sc_edges


def kernel(user_emb, item_emb, edge_index, edge_values, W1, b1, W2, b2):
    node_emb = jnp.concatenate([user_emb, item_emb], axis=0)
    half = edge_index.shape[1] // 2
    src = edge_index[0, :half]
    dst = edge_index[1, :half]

    P, Q = _make_tables(node_emb, W1, b1)

    bias = 0.0 + 0.0001
    eps_key = jax.random.key(42)
    eps = (bias - (1.0 - bias)) * jax.random.uniform(
        eps_key, (half, 1), dtype=jnp.float32
    ) + (1.0 - bias)
    noise = jnp.log(eps) - jnp.log(1.0 - eps)
    n2 = (noise[:, 0] + b2[0]) / _B_TEMP

    per_tile_chunks = -(-half // (_NT * _C))
    e_pad = _NT * per_tile_chunks * _C
    padn = e_pad - half

    srcp = jnp.pad(src, (0, padn)).reshape(_NT, per_tile_chunks, _C)
    dstp = jnp.pad(dst, (0, padn)).reshape(_NT, per_tile_chunks, _C)
    evp = jnp.pad(edge_values[:half], (0, padn)).reshape(
        _NT, per_tile_chunks, _C
    )
    n2p = jnp.pad(n2, (0, padn), constant_values=-100.0).reshape(
        _NT, per_tile_chunks, _C
    )
    w2x = W2[:, 0] / _B_TEMP

    outv, accp = _make_sc_kernel(per_tile_chunks)(
        P, Q, srcp, dstp, evp, n2p, w2x
    )

    new_vals = outv.reshape(-1)[:half]
    mean_edge_weight = jnp.sum(accp) / half
    sym_vals = jnp.concatenate([new_vals, new_vals])
    sym_rows = jnp.concatenate([src, dst])
    sym_cols = jnp.concatenate([dst, src])
    return sym_vals, sym_rows, sym_cols, mean_edge_weight
